# Initial kernel scaffold; baseline (speedup 1.0000x reference)
import jax
import jax.numpy as jnp
from jax.experimental import pallas as pl


def kernel(x, alpha):
    raise NotImplementedError("write your pallas kernel here")


# speedup vs baseline: 17521.4889x; 17521.4889x over previous
import math

import jax
import jax.numpy as jnp
from jax.experimental import pallas as pl
from jax.experimental.pallas import tpu as pltpu

_F32 = jnp.float32
_I32 = jnp.int32

_T0 = int(jnp.float32(0.75 * 2.0 ** -10).view(jnp.int32)) if False else None


def _bits_const(v):
    import numpy as np
    return int(np.float32(v).view(np.int32))


_T0_BITS = _bits_const(0.75 * 2.0 ** -10)
_T1_BITS = _bits_const(1.75 * 2.0 ** -10)
_MID0 = 0.75 * 2.0 ** -10


def _quantize_block(x, a):
    bc = jax.lax.bitcast_convert_type
    xn = jnp.clip(x, -a, a) / a
    nb = bc(xn, _I32)
    neg = jax.lax.shift_right_logical(nb, 31)
    ubits = (nb & jnp.int32(0x7FFFFFFF)) + jnp.int32(1 << 23)
    ubx = ubits + neg
    c0 = ubx > jnp.int32(_T0_BITS)
    c1 = ubx > jnp.int32(_T1_BITS)
    e = jnp.minimum(jax.lax.shift_right_logical(ubits, 23) - 127, 0)
    scale = bc(jax.lax.shift_left((127 - e), 23), _F32)
    f = bc(ubits, _F32) * scale - 1.0
    fbits = bc(f, _I32)
    c2 = (fbits + neg) > jax.lax.shift_left(115 - e, 23)
    ef = jax.lax.shift_right_logical(fbits, 23) - 127
    mant = fbits & jnp.int32(0x7FFFFF)
    mt = ((mant + neg) > jnp.int32(0x400000)).astype(_I32)
    gexp = jnp.maximum(ef + mt, -11 - e)
    p1 = bc(jax.lax.shift_left(126 + e, 23), _F32)
    p2 = bc(jax.lax.shift_left(126 + e + gexp, 23), _F32)
    mag = p1 + jnp.where(c2, p2, jnp.float32(0.0))
    mag = jnp.where(c1, mag, jnp.float32(_MID0))
    mag = jnp.where(c0, mag, jnp.float32(0.0))
    ob = bc(mag, _I32) | (nb & jnp.int32(-2147483648))
    return bc(ob, _F32) * a


def _tc_body(a_ref, x_ref, o_ref):
    o_ref[...] = _quantize_block(x_ref[...], a_ref[0])


def kernel(x, alpha):
    g = jnp.float32(1.0 / math.sqrt(x.size))
    ag = alpha * g
    a_s = (alpha - ag) + ag
    a_s = jnp.maximum(a_s, jnp.float32(1e-8))

    rows = x.shape[0] * x.shape[1]
    cols = x.shape[2]
    x2 = x.reshape(rows, cols)
    block_rows = 256
    grid = rows // block_rows
    out = pl.pallas_call(
        _tc_body,
        grid=(grid,),
        in_specs=[
            pl.BlockSpec(memory_space=pltpu.SMEM),
            pl.BlockSpec((block_rows, cols), lambda i: (i, 0)),
        ],
        out_specs=pl.BlockSpec((block_rows, cols), lambda i: (i, 0)),
        out_shape=jax.ShapeDtypeStruct((rows, cols), jnp.float32),
        compiler_params=pltpu.CompilerParams(
            dimension_semantics=("arbitrary",),
        ),
    )(a_s.reshape(1), x2)
    return out.reshape(x.shape)


# speedup vs baseline: 22778.7470x; 1.3000x over previous
import math

import jax
import jax.numpy as jnp
from jax.experimental import pallas as pl
from jax.experimental.pallas import tpu as pltpu

_F32 = jnp.float32
_I32 = jnp.int32

_T0 = int(jnp.float32(0.75 * 2.0 ** -10).view(jnp.int32)) if False else None


def _bits_const(v):
    import numpy as np
    return int(np.float32(v).view(np.int32))


_T0_BITS = _bits_const(0.75 * 2.0 ** -10)
_T1_BITS = _bits_const(1.75 * 2.0 ** -10)
_MID0 = 0.75 * 2.0 ** -10


def _quantize_block(x, a, rcp):
    bc = jax.lax.bitcast_convert_type
    srl = jax.lax.shift_right_logical
    sll = jax.lax.shift_left
    xb = bc(x, _I32)
    neg = srl(xb, 31)
    ax = bc(xb & jnp.int32(0x7FFFFFFF), _F32)
    t = jnp.minimum(ax * rcp, jnp.float32(0.99999994))
    ubits = bc(t, _I32) + jnp.int32(1 << 23)
    ubx = ubits + neg
    c0 = ubx > jnp.int32(_T0_BITS)
    c1 = ubx > jnp.int32(_T1_BITS)
    E = srl(ubits, 23)
    scale = bc(sll(254 - E, 23), _F32)
    f = bc(ubits, _F32) * scale - 1.0
    fbits = bc(f, _I32)
    fbx = fbits + neg
    c2 = fbx > sll(242 - E, 23)
    rb23 = srl(fbx + jnp.int32(0x3FFFFF), 23)
    p1 = bc(sll(E - 1, 23), _F32)
    p2 = bc(sll((E - 128) + rb23, 23), _F32)
    p2 = jnp.maximum(p2, jnp.float32(2.0 ** -12))
    mag = p1 + jnp.where(c2, p2, jnp.float32(0.0))
    mag = jnp.where(c1, mag, jnp.float32(_MID0))
    mag = jnp.where(c0, mag, jnp.float32(0.0))
    ob = bc(mag, _I32) | (xb & jnp.int32(-2147483648))
    return bc(ob, _F32) * a


def _tc_body(a_ref, x_ref, o_ref):
    o_ref[...] = _quantize_block(x_ref[...], a_ref[0], a_ref[1])


def kernel(x, alpha):
    g = jnp.float32(1.0 / math.sqrt(x.size))
    ag = alpha * g
    a_s = (alpha - ag) + ag
    a_s = jnp.maximum(a_s, jnp.float32(1e-8))
    rcp = jnp.float32(1.0) / a_s

    rows = x.shape[0] * x.shape[1]
    cols = x.shape[2]
    x2 = x.reshape(rows, cols)
    block_rows = 256
    grid = rows // block_rows
    out = pl.pallas_call(
        _tc_body,
        grid=(grid,),
        in_specs=[
            pl.BlockSpec(memory_space=pltpu.SMEM),
            pl.BlockSpec((block_rows, cols), lambda i: (i, 0)),
        ],
        out_specs=pl.BlockSpec((block_rows, cols), lambda i: (i, 0)),
        out_shape=jax.ShapeDtypeStruct((rows, cols), jnp.float32),
        compiler_params=pltpu.CompilerParams(
            dimension_semantics=("arbitrary",),
        ),
    )(jnp.stack([a_s, rcp]), x2)
    return out.reshape(x.shape)


# speedup vs baseline: 26910.3112x; 1.1814x over previous
import math

import jax
import jax.numpy as jnp
from jax.experimental import pallas as pl
from jax.experimental.pallas import tpu as pltpu

_F32 = jnp.float32
_I32 = jnp.int32

_T0 = int(jnp.float32(0.75 * 2.0 ** -10).view(jnp.int32)) if False else None


def _bits_const(v):
    import numpy as np
    return int(np.float32(v).view(np.int32))


_T0_BITS = _bits_const(0.75 * 2.0 ** -11)
_T1_BITS = _bits_const(1.75 * 2.0 ** -11)
_MID0 = 1.5 * 2.0 ** -11


def _quantize_block(x, a, rcp):
    bc = jax.lax.bitcast_convert_type
    srl = jax.lax.shift_right_logical
    sra = jax.lax.shift_right_arithmetic
    sll = jax.lax.shift_left
    xb = bc(x, _I32)
    neg = srl(xb, 31)
    ax = bc(xb & jnp.int32(0x7FFFFFFF), _F32)
    t = jnp.minimum(ax * rcp, jnp.float32(0.99999994))
    tb = bc(t, _I32)
    tbx = tb + neg
    c0 = tbx > jnp.int32(_T0_BITS)
    c1 = tbx > jnp.int32(_T1_BITS)
    Et = srl(tb, 23)
    M = tb & jnp.int32(0x7FFFFF)
    fM = M.astype(_F32)
    cbx = bc(fM, _I32) + neg
    p1b = sll(Et, 23)
    ct = bc(cbx, jnp.uint32) + bc(p1b, jnp.uint32)
    c2 = ct > jnp.uint32(264 << 23)
    rbm = sra(cbx + jnp.int32(0x3FFFFF - (150 << 23)), 23)
    p2 = bc(sll(Et + rbm, 23), _F32)
    p2 = jnp.maximum(p2, jnp.float32(2.0 ** -12))
    mag = bc(p1b, _F32) + jnp.where(c2, p2, jnp.float32(0.0))
    mag = jnp.where(c1, mag, jnp.float32(_MID0))
    mag = jnp.where(c0, mag, jnp.float32(0.0))
    ob = bc(mag, _I32) | (xb & jnp.int32(-2147483648))
    return bc(ob, _F32) * a


def _tc_body(a_ref, x_ref, o_ref):
    o_ref[...] = _quantize_block(x_ref[...], a_ref[0], a_ref[1])


def kernel(x, alpha):
    g = jnp.float32(1.0 / math.sqrt(x.size))
    ag = alpha * g
    a_s = (alpha - ag) + ag
    a_s = jnp.maximum(a_s, jnp.float32(1e-8))
    rcp = jnp.float32(1.0) / a_s

    rows = x.shape[0] * x.shape[1]
    cols = x.shape[2]
    x2 = x.reshape(rows, cols)
    block_rows = 256
    grid = rows // block_rows
    out = pl.pallas_call(
        _tc_body,
        grid=(grid,),
        in_specs=[
            pl.BlockSpec(memory_space=pltpu.SMEM),
            pl.BlockSpec((block_rows, cols), lambda i: (i, 0)),
        ],
        out_specs=pl.BlockSpec((block_rows, cols), lambda i: (i, 0)),
        out_shape=jax.ShapeDtypeStruct((rows, cols), jnp.float32),
        compiler_params=pltpu.CompilerParams(
            dimension_semantics=("arbitrary",),
        ),
    )(jnp.stack([a_s, rcp]), x2)
    return out.reshape(x.shape)
